# Initial kernel scaffold; baseline (speedup 1.0000x reference)
import jax
import jax.numpy as jnp
from jax.experimental import pallas as pl


def kernel(x, edge_index, W, b):
    raise NotImplementedError("write your pallas kernel here")


# speedup vs baseline: 7.6151x; 7.6151x over previous
import functools

import jax
import jax.numpy as jnp
from jax import lax
from jax.experimental import pallas as pl
from jax.experimental.pallas import tpu as pltpu
from jax.experimental.pallas import tpu_sc as plsc

N = 10000
E = 320000
D = 128

NC = 2
NS = 16
NW = NC * NS
EPW = E // NW
CHUNK = 80
NCHUNK = EPW // CHUNK
RPS = 632
RPS_LAST = N - 15 * RPS


def _segsum_sc(x, src, dst):
    mesh = plsc.VectorSubcoreMesh(core_axis_name="c", subcore_axis_name="s")

    @functools.partial(
        pl.kernel,
        mesh=mesh,
        out_type=jax.ShapeDtypeStruct((2 * N, D), jnp.float32),
        scratch_types=[
            pltpu.VMEM((NCHUNK, CHUNK), jnp.int32),
            pltpu.VMEM((NCHUNK, CHUNK), jnp.int32),
            pltpu.VMEM((CHUNK, D), jnp.float32),
            pltpu.VMEM_SHARED((N, D), jnp.float32),
            pltpu.SemaphoreType.DMA,
        ],
    )
    def k(x_hbm, src_hbm, dst_hbm, out_hbm, src_v, dst_v, rows_v, acc, sem):
        cid = lax.axis_index("c")
        sid = lax.axis_index("s")
        wid = cid * NS + sid

        zeros16 = jnp.zeros((16,), jnp.float32)

        def zero_body(t, _):
            rows_v[t // (D // 16), pl.ds((t % (D // 16)) * 16, 16)] = zeros16
            return _

        lax.fori_loop(0, CHUNK * (D // 16), zero_body, None)
        row0 = pl.multiple_of(sid * RPS, 8)

        @pl.when(sid < NS - 1)
        def _():
            for j in range(RPS // CHUNK):
                pltpu.sync_copy(rows_v, acc.at[pl.ds(row0 + j * CHUNK, CHUNK)])
            rem = RPS % CHUNK
            if rem:
                pltpu.sync_copy(
                    rows_v.at[pl.ds(0, rem)],
                    acc.at[pl.ds(row0 + (RPS // CHUNK) * CHUNK, rem)])

        @pl.when(sid == NS - 1)
        def _():
            base = (NS - 1) * RPS
            for j in range(RPS_LAST // CHUNK):
                pltpu.sync_copy(rows_v, acc.at[pl.ds(base + j * CHUNK, CHUNK)])
            rem = RPS_LAST % CHUNK
            if rem:
                pltpu.sync_copy(
                    rows_v.at[pl.ds(0, rem)],
                    acc.at[pl.ds(base + (RPS_LAST // CHUNK) * CHUNK, rem)])

        pltpu.sync_copy(src_hbm.at[wid], src_v)
        pltpu.sync_copy(dst_hbm.at[wid], dst_v)

        plsc.subcore_barrier()

        def edge_body(i, _):
            pltpu.async_copy(x_hbm.at[src_v.at[i]], rows_v, sem).wait()
            pltpu.sync_copy(rows_v, acc.at[dst_v.at[i]], add=True)
            return _

        lax.fori_loop(0, NCHUNK, edge_body, None)

        plsc.subcore_barrier()

        out0 = pl.multiple_of(cid * N + sid * RPS, 8)

        @pl.when(sid < NS - 1)
        def _():
            pltpu.sync_copy(acc.at[pl.ds(row0, RPS)],
                            out_hbm.at[pl.ds(out0, RPS)])

        @pl.when(sid == NS - 1)
        def _():
            pltpu.sync_copy(
                acc.at[pl.ds((NS - 1) * RPS, RPS_LAST)],
                out_hbm.at[pl.ds(cid * N + (NS - 1) * RPS, RPS_LAST)],
            )

    return k(x, src, dst)


def _mm_kernel(p0_ref, p1_ref, w_ref, b_ref, o_ref):
    s = p0_ref[...] + p1_ref[...]
    y = jnp.dot(s, w_ref[...], preferred_element_type=jnp.float32,
                precision=jax.lax.Precision.HIGHEST)
    o_ref[...] = jnp.maximum(y + b_ref[...], 0.0)


def _finish_tc(partials, W, b2):
    blk = 1000
    nblk = N // blk
    return pl.pallas_call(
        _mm_kernel,
        grid=(nblk,),
        in_specs=[
            pl.BlockSpec((blk, D), lambda i: (i, 0)),
            pl.BlockSpec((blk, D), lambda i: (i + nblk, 0)),
            pl.BlockSpec((D, D), lambda i: (0, 0)),
            pl.BlockSpec((1, D), lambda i: (0, 0)),
        ],
        out_specs=pl.BlockSpec((blk, D), lambda i: (i, 0)),
        out_shape=jax.ShapeDtypeStruct((N, D), jnp.float32),
    )(partials, partials, W, b2)


def kernel(x, edge_index, W, b):
    ei = edge_index.astype(jnp.int32)
    src = ei[0].reshape(NW, NCHUNK, CHUNK)
    dst = ei[1].reshape(NW, NCHUNK, CHUNK)
    partials = _segsum_sc(x, src, dst)
    return _finish_tc(partials, W, b.reshape(1, D))
